# baseline (speedup 1.0000x reference)
import jax
import jax.numpy as jnp
from jax.experimental import pallas as pl
from jax.experimental.pallas import tpu as pltpu


def _dma_copy(x_ref, o_ref, sem):
    copy = pltpu.make_async_copy(x_ref, o_ref, sem)
    copy.start()
    copy.wait()


def _copy_2d(x):
    b, t = x.shape
    return pl.pallas_call(
        _dma_copy,
        out_shape=jax.ShapeDtypeStruct((b, t), x.dtype),
        in_specs=[pl.BlockSpec(memory_space=pl.ANY)],
        out_specs=pl.BlockSpec(memory_space=pl.ANY),
        scratch_shapes=[pltpu.SemaphoreType.DMA],
    )(x)


def kernel(x):
    squeeze_batch = False
    if x.ndim == 1:
        x = x[None, :]
        squeeze_batch = True
    out = _copy_2d(x)
    if squeeze_batch:
        out = out[0]
    return out


# speedup vs baseline: 37.9781x; 37.9781x over previous
import jax
import jax.numpy as jnp
from jax.experimental import pallas as pl
from jax.experimental.pallas import tpu as pltpu


def _copy_block(x_ref, o_ref):
    o_ref[...] = x_ref[...]


def _copy_2d(x):
    b, t = x.shape
    blk = 16384
    if t % blk != 0:
        blk = t
    grid = t // blk
    return pl.pallas_call(
        _copy_block,
        out_shape=jax.ShapeDtypeStruct((b, t), x.dtype),
        grid=(grid,),
        in_specs=[pl.BlockSpec((b, blk), lambda i: (0, i))],
        out_specs=pl.BlockSpec((b, blk), lambda i: (0, i)),
    )(x)


def kernel(x):
    squeeze_batch = False
    if x.ndim == 1:
        x = x[None, :]
        squeeze_batch = True
    out = _copy_2d(x)
    if squeeze_batch:
        out = out[0]
    return out


# speedup vs baseline: 42.6849x; 1.1239x over previous
import jax
import jax.numpy as jnp
from jax.experimental import pallas as pl
from jax.experimental.pallas import tpu as pltpu


def _copy_block(x_ref, o_ref):
    o_ref[...] = x_ref[...]


def _copy_2d(x):
    b, t = x.shape
    blk = 32768
    if t % blk != 0:
        blk = t
    grid = t // blk
    return pl.pallas_call(
        _copy_block,
        out_shape=jax.ShapeDtypeStruct((b, t), x.dtype),
        grid=(grid,),
        in_specs=[pl.BlockSpec((b, blk), lambda i: (0, i))],
        out_specs=pl.BlockSpec((b, blk), lambda i: (0, i)),
    )(x)


def kernel(x):
    squeeze_batch = False
    if x.ndim == 1:
        x = x[None, :]
        squeeze_batch = True
    out = _copy_2d(x)
    if squeeze_batch:
        out = out[0]
    return out


# speedup vs baseline: 47.2433x; 1.1068x over previous
import jax
import jax.numpy as jnp
from jax.experimental import pallas as pl
from jax.experimental.pallas import tpu as pltpu


def _copy_block(x_ref, o_ref):
    o_ref[...] = x_ref[...]


def _copy_2d(x):
    b, t = x.shape
    blk = 65536
    if t % blk != 0:
        blk = t
    grid = t // blk
    return pl.pallas_call(
        _copy_block,
        out_shape=jax.ShapeDtypeStruct((b, t), x.dtype),
        grid=(grid,),
        in_specs=[pl.BlockSpec((b, blk), lambda i: (0, i))],
        out_specs=pl.BlockSpec((b, blk), lambda i: (0, i)),
    )(x)


def kernel(x):
    squeeze_batch = False
    if x.ndim == 1:
        x = x[None, :]
        squeeze_batch = True
    out = _copy_2d(x)
    if squeeze_batch:
        out = out[0]
    return out


# speedup vs baseline: 47.6323x; 1.0082x over previous
import jax
import jax.numpy as jnp
from jax.experimental import pallas as pl
from jax.experimental.pallas import tpu as pltpu


def _copy_block(x_ref, o_ref):
    o_ref[...] = x_ref[...]


def _copy_2d(x):
    b, t = x.shape
    rblk = 16
    if b % rblk != 0:
        rblk = b
    grid = b // rblk
    return pl.pallas_call(
        _copy_block,
        out_shape=jax.ShapeDtypeStruct((b, t), x.dtype),
        grid=(grid,),
        in_specs=[pl.BlockSpec((rblk, t), lambda i: (i, 0))],
        out_specs=pl.BlockSpec((rblk, t), lambda i: (i, 0)),
    )(x)


def kernel(x):
    squeeze_batch = False
    if x.ndim == 1:
        x = x[None, :]
        squeeze_batch = True
    out = _copy_2d(x)
    if squeeze_batch:
        out = out[0]
    return out
